# Initial kernel scaffold; baseline (speedup 1.0000x reference)
import jax
import jax.numpy as jnp
from jax.experimental import pallas as pl


def kernel(x, edge_index, batch, W0, a_s0, a_d0, b0, W1, a_s1, a_d1, b1, W2, a_s2, a_d2, b2):
    raise NotImplementedError("write your pallas kernel here")


# speedup vs baseline: 40.9109x; 40.9109x over previous
import jax
import jax.numpy as jnp
from jax import lax
from jax.experimental import pallas as pl
from jax.experimental.pallas import tpu as pltpu
from jax.experimental.pallas import tpu_sc as plsc

N = 10000
D = 128
H = 4
C = 32
HC = H * C
OUT = 64
G = 64
E = 320000
E1 = E + N
NEG = 0.2

NC = 2
NS = 16
NW = NC * NS
K = 128
EW = 10368
E_PAD = NW * EW
N_PAD = 10240
SLICE = N_PAD // NS
HP = 16
RB = 1280
GRID = N_PAD // RB


def _leaky(v):
    return jnp.where(v > 0, v, NEG * v)


def _make_sc_edge(Hh, F):
    CW = F // Hh
    CH = EW // K
    EPI = 16 // Hh

    def body(src_hbm, dst_hbm, ab_hbm, h_hbm, out_hbm, den_hbm,
             acc_out, acc_den, src_v, dst_v, sa, sb, hrows, s_rows,
             sem_h, sem_a, sem_b):
        cid = lax.axis_index("c")
        sid = lax.axis_index("s")
        wid = sid * NC + cid
        lanes = lax.iota(jnp.int32, 16)

        def zr(k_, _):
            for c0 in range(0, F, 16):
                hrows[k_, pl.ds(c0, 16)] = jnp.zeros((16,), jnp.float32)
            return 0
        lax.fori_loop(0, K, zr, 0)

        def zs(i, _):
            plsc.store_scatter(s_rows, [jnp.full((16,), i, jnp.int32), lanes],
                               jnp.zeros((16,), jnp.float32))
            return 0
        lax.fori_loop(0, K, zs, 0)

        base = sid * SLICE
        for j in range(SLICE // K):
            pltpu.sync_copy(hrows, acc_out.at[pl.ds(base + j * K, K)])
            pltpu.sync_copy(s_rows, acc_den.at[pl.ds(base + j * K, K)])
        plsc.subcore_barrier()

        ebase = wid * EW

        def chunk(i, _):
            off = ebase + i * K
            pltpu.sync_copy(src_hbm.at[pl.ds(off, K)], src_v)
            pltpu.sync_copy(dst_hbm.at[pl.ds(off, K)], dst_v)
            gh = pltpu.async_copy(h_hbm.at[src_v], hrows, sem_h)
            ga = pltpu.async_copy(ab_hbm.at[src_v], sa, sem_a)
            gb = pltpu.async_copy(ab_hbm.at[dst_v], sb, sem_b)
            ga.wait()
            gb.wait()

            def sgrp(j_, _):
                rows = j_ * EPI + lanes // Hh
                cols = lanes % Hh
                a = plsc.load_gather(sa, [rows, cols])
                b = plsc.load_gather(sb, [rows, cols + 8])
                sv = jnp.exp(_leaky(a + b))
                plsc.store_scatter(s_rows, [rows, cols], sv)
                return 0
            lax.fori_loop(0, K // EPI, sgrp, 0)
            gh.wait()

            def erow(k_, _):
                row_idx = jnp.full((16,), k_, jnp.int32)
                for hh in range(Hh):
                    m = plsc.load_gather(
                        s_rows, [row_idx, jnp.full((16,), hh, jnp.int32)])
                    for q in range(CW // 16):
                        c0 = hh * CW + q * 16
                        hrows[k_, pl.ds(c0, 16)] = hrows[k_, pl.ds(c0, 16)] * m
                return 0
            lax.fori_loop(0, K, erow, 0)

            pltpu.sync_copy(hrows, acc_out.at[dst_v], add=True)
            pltpu.sync_copy(s_rows, acc_den.at[dst_v], add=True)
            return 0
        lax.fori_loop(0, CH, chunk, 0)
        plsc.subcore_barrier()

        pltpu.sync_copy(acc_out.at[pl.ds(base, SLICE)],
                        out_hbm.at[cid, pl.ds(base, SLICE)])
        pltpu.sync_copy(acc_den.at[pl.ds(base, SLICE)],
                        den_hbm.at[cid, pl.ds(base, SLICE)])

    return pl.kernel(
        body,
        out_type=(jax.ShapeDtypeStruct((NC, N_PAD, F), jnp.float32),
                  jax.ShapeDtypeStruct((NC, N_PAD, HP), jnp.float32)),
        mesh=plsc.VectorSubcoreMesh(core_axis_name="c", subcore_axis_name="s"),
        compiler_params=pltpu.CompilerParams(needs_layout_passes=False,
                                             use_tc_tiling_on_sc=False),
        scratch_types=(
            pltpu.VMEM_SHARED((N_PAD, F), jnp.float32),
            pltpu.VMEM_SHARED((N_PAD, HP), jnp.float32),
            pltpu.VMEM((K,), jnp.int32),
            pltpu.VMEM((K,), jnp.int32),
            pltpu.VMEM((K, HP), jnp.float32),
            pltpu.VMEM((K, HP), jnp.float32),
            pltpu.VMEM((K, F), jnp.float32),
            pltpu.VMEM((K, HP), jnp.float32),
            pltpu.SemaphoreType.DMA,
            pltpu.SemaphoreType.DMA,
            pltpu.SemaphoreType.DMA,
        ),
    )


_sc_edge4 = _make_sc_edge(H, HC)
_sc_edge1 = _make_sc_edge(1, OUT)


def _tc_head(xp, W, Asd):
    Fin, Fout = W.shape

    def body(x_ref, w_ref, p_ref, h_ref, ab_ref):
        h = jnp.dot(x_ref[...], w_ref[...], preferred_element_type=jnp.float32)
        h_ref[...] = h
        ab_ref[...] = jnp.dot(h, p_ref[...], preferred_element_type=jnp.float32)

    return pl.pallas_call(
        body,
        grid=(GRID,),
        in_specs=[pl.BlockSpec((RB, Fin), lambda i: (i, 0)),
                  pl.BlockSpec((Fin, Fout), lambda i: (0, 0)),
                  pl.BlockSpec((Fout, HP), lambda i: (0, 0))],
        out_specs=[pl.BlockSpec((RB, Fout), lambda i: (i, 0)),
                   pl.BlockSpec((RB, HP), lambda i: (i, 0))],
        out_shape=[jax.ShapeDtypeStruct((N_PAD, Fout), jnp.float32),
                   jax.ShapeDtypeStruct((N_PAD, HP), jnp.float32)],
    )(xp, W, Asd)


def _tc_mid(oA, oB, dA, dB, expand, bias, W, Asd):
    Hh_in = expand.shape[0]
    Fin, Fout = W.shape

    def body(oa, ob, da, db, ex, bi, w, p_ref, h_ref, ab_ref):
        den = da[...] + db[...]
        dex = jnp.dot(den[:, :Hh_in], ex[...],
                      preferred_element_type=jnp.float32) + 1e-16
        g = _leaky((oa[...] + ob[...]) / dex + bi[...])
        h = jnp.dot(g, w[...], preferred_element_type=jnp.float32)
        h_ref[...] = h
        ab_ref[...] = jnp.dot(h, p_ref[...], preferred_element_type=jnp.float32)

    return pl.pallas_call(
        body,
        grid=(GRID,),
        in_specs=[pl.BlockSpec((RB, Fin), lambda i: (i, 0)),
                  pl.BlockSpec((RB, Fin), lambda i: (i, 0)),
                  pl.BlockSpec((RB, HP), lambda i: (i, 0)),
                  pl.BlockSpec((RB, HP), lambda i: (i, 0)),
                  pl.BlockSpec((Hh_in, Fin), lambda i: (0, 0)),
                  pl.BlockSpec((1, Fin), lambda i: (0, 0)),
                  pl.BlockSpec((Fin, Fout), lambda i: (0, 0)),
                  pl.BlockSpec((Fout, HP), lambda i: (0, 0))],
        out_specs=[pl.BlockSpec((RB, Fout), lambda i: (i, 0)),
                   pl.BlockSpec((RB, HP), lambda i: (i, 0))],
        out_shape=[jax.ShapeDtypeStruct((N_PAD, Fout), jnp.float32),
                   jax.ShapeDtypeStruct((N_PAD, HP), jnp.float32)],
    )(oA, oB, dA, dB, expand, bias, W, Asd)


def _tc_pool(oA, oB, dA, dB, bias, batch2d):
    def body(oa, ob, da, db, bi, bt, out_ref, acc, cnt):
        i = pl.program_id(0)
        den = (da[...] + db[...])[:, :1]
        dex = jnp.dot(den, jnp.ones((1, OUT), jnp.float32),
                      preferred_element_type=jnp.float32) + 1e-16
        h = _leaky((oa[...] + ob[...]) / dex + bi[...])
        onehot_t = (lax.broadcasted_iota(jnp.int32, (G, RB), 0)
                    == bt[...]).astype(jnp.float32)
        sums = jnp.dot(onehot_t, h, preferred_element_type=jnp.float32)
        counts = jnp.dot(onehot_t, jnp.ones((RB, 1), jnp.float32),
                         preferred_element_type=jnp.float32)

        @pl.when(i == 0)
        def _():
            acc[...] = jnp.zeros_like(acc)
            cnt[...] = jnp.zeros_like(cnt)

        acc[...] += sums
        cnt[...] += counts

        @pl.when(i == GRID - 1)
        def _():
            out_ref[...] = acc[...] / jnp.maximum(cnt[...], 1.0)

    return pl.pallas_call(
        body,
        grid=(GRID,),
        in_specs=[pl.BlockSpec((RB, OUT), lambda i: (i, 0)),
                  pl.BlockSpec((RB, OUT), lambda i: (i, 0)),
                  pl.BlockSpec((RB, HP), lambda i: (i, 0)),
                  pl.BlockSpec((RB, HP), lambda i: (i, 0)),
                  pl.BlockSpec((1, OUT), lambda i: (0, 0)),
                  pl.BlockSpec((1, RB), lambda i: (0, i))],
        out_specs=pl.BlockSpec((G, OUT), lambda i: (0, 0)),
        out_shape=jax.ShapeDtypeStruct((G, OUT), jnp.float32),
        scratch_shapes=[pltpu.VMEM((G, OUT), jnp.float32),
                        pltpu.VMEM((G, 1), jnp.float32)],
    )(oA, oB, dA, dB, bias, batch2d)


def _proj(a_s, a_d, Fout, Hh):
    if Hh > 1:
        mask = jnp.repeat(jnp.eye(Hh, dtype=jnp.float32), Fout // Hh, axis=0)
        As = a_s.reshape(Fout, 1) * mask
        Ad = a_d.reshape(Fout, 1) * mask
    else:
        As = a_s.reshape(Fout, 1)
        Ad = a_d.reshape(Fout, 1)
    z = jnp.zeros((Fout, 8 - Hh), jnp.float32)
    return jnp.concatenate([As, z, Ad, z], axis=1)


def kernel(x, edge_index, batch, W0, a_s0, a_d0, b0,
           W1, a_s1, a_d1, b1, W2, a_s2, a_d2, b2):
    f32 = jnp.float32
    ei = edge_index.astype(jnp.int32)
    loops = jnp.arange(N, dtype=jnp.int32)
    pad_e = jnp.full((E_PAD - E1,), N, jnp.int32)
    src = jnp.concatenate([ei[0], loops, pad_e])
    dst = jnp.concatenate([ei[1], loops, pad_e])
    xp = jnp.pad(x.astype(f32), ((0, N_PAD - N), (0, 0)))
    batch2d = jnp.pad(batch.astype(jnp.int32), (0, N_PAD - N),
                      constant_values=G).reshape(1, N_PAD)

    mask4 = jnp.repeat(jnp.eye(H, dtype=f32), C, axis=0)
    expand4 = mask4.T
    P0 = _proj(a_s0, a_d0, HC, H)
    P1 = _proj(a_s1, a_d1, HC, H)
    P2 = _proj(a_s2, a_d2, OUT, 1)

    h0, ab0 = _tc_head(xp, W0, P0)
    o0, den0 = _sc_edge4(src, dst, ab0, h0)
    h1, ab1 = _tc_mid(o0[0], o0[1], den0[0], den0[1], expand4,
                      b0.reshape(1, HC), W1, P1)
    o1, den1 = _sc_edge4(src, dst, ab1, h1)
    h2, ab2 = _tc_mid(o1[0], o1[1], den1[0], den1[1], expand4,
                      b1.reshape(1, HC), W2, P2)
    o2, den2 = _sc_edge1(src, dst, ab2, h2)
    return _tc_pool(o2[0], o2[1], den2[0], den2[1],
                    b2.reshape(1, OUT), batch2d)


# speedup vs baseline: 41.1663x; 1.0062x over previous
import jax
import jax.numpy as jnp
from jax import lax
from jax.experimental import pallas as pl
from jax.experimental.pallas import tpu as pltpu
from jax.experimental.pallas import tpu_sc as plsc

N = 10000
D = 128
H = 4
C = 32
HC = H * C
OUT = 64
G = 64
E = 320000
E1 = E + N
NEG = 0.2

NC = 2
NS = 16
NW = NC * NS
K = 128
EW = 10368
E_PAD = NW * EW
N_PAD = 10240
SLICE = N_PAD // NS
HP = 16
RB = 1280
GRID = N_PAD // RB


def _leaky(v):
    return jnp.where(v > 0, v, NEG * v)


def _make_sc_edge(Hh, F):
    CW = F // Hh
    CH = EW // K
    EPI = 16 // Hh

    def body(src_hbm, dst_hbm, ab_hbm, h_hbm, out_hbm, den_hbm,
             acc_out, acc_den, src_v, dst_v, sa, sb, hrows, s_rows,
             sem_h, sem_a, sem_b):
        cid = lax.axis_index("c")
        sid = lax.axis_index("s")
        wid = sid * NC + cid
        lanes = lax.iota(jnp.int32, 16)

        def zr(k_, _):
            for c0 in range(0, F, 16):
                hrows[k_, pl.ds(c0, 16)] = jnp.zeros((16,), jnp.float32)
            return 0
        lax.fori_loop(0, K, zr, 0)

        def zs(i, _):
            plsc.store_scatter(s_rows, [jnp.full((16,), i, jnp.int32), lanes],
                               jnp.zeros((16,), jnp.float32))
            return 0
        lax.fori_loop(0, K, zs, 0)

        base = sid * SLICE
        for j in range(SLICE // K):
            pltpu.sync_copy(hrows, acc_out.at[pl.ds(base + j * K, K)])
            pltpu.sync_copy(s_rows, acc_den.at[pl.ds(base + j * K, K)])
        plsc.subcore_barrier()

        ebase = wid * EW

        def chunk(i, _):
            off = ebase + i * K
            pltpu.sync_copy(src_hbm.at[pl.ds(off, K)], src_v)
            pltpu.sync_copy(dst_hbm.at[pl.ds(off, K)], dst_v)
            gh = pltpu.async_copy(h_hbm.at[src_v], hrows, sem_h)
            ga = pltpu.async_copy(ab_hbm.at[src_v], sa, sem_a)
            gb = pltpu.async_copy(ab_hbm.at[dst_v], sb, sem_b)
            ga.wait()
            gb.wait()

            def sgrp(j_, _):
                rows = j_ * EPI + lanes // Hh
                cols = lanes % Hh
                a = plsc.load_gather(sa, [rows, cols])
                b = plsc.load_gather(sb, [rows, cols + 8])
                sv = jnp.exp(_leaky(a + b))
                plsc.store_scatter(s_rows, [rows, cols], sv)
                return 0
            lax.fori_loop(0, K // EPI, sgrp, 0)
            gh.wait()

            def erow(k_, _):
                row_idx = jnp.full((16,), k_, jnp.int32)
                for hh in range(Hh):
                    m = plsc.load_gather(
                        s_rows, [row_idx, jnp.full((16,), hh, jnp.int32)])
                    for q in range(CW // 16):
                        c0 = hh * CW + q * 16
                        hrows[k_, pl.ds(c0, 16)] = hrows[k_, pl.ds(c0, 16)] * m
                return 0
            lax.fori_loop(0, K, erow, 0, unroll=4)

            pltpu.sync_copy(hrows, acc_out.at[dst_v], add=True)
            pltpu.sync_copy(s_rows, acc_den.at[dst_v], add=True)
            return 0
        lax.fori_loop(0, CH, chunk, 0)
        plsc.subcore_barrier()

        pltpu.sync_copy(acc_out.at[pl.ds(base, SLICE)],
                        out_hbm.at[cid, pl.ds(base, SLICE)])
        pltpu.sync_copy(acc_den.at[pl.ds(base, SLICE)],
                        den_hbm.at[cid, pl.ds(base, SLICE)])

    return pl.kernel(
        body,
        out_type=(jax.ShapeDtypeStruct((NC, N_PAD, F), jnp.float32),
                  jax.ShapeDtypeStruct((NC, N_PAD, HP), jnp.float32)),
        mesh=plsc.VectorSubcoreMesh(core_axis_name="c", subcore_axis_name="s"),
        compiler_params=pltpu.CompilerParams(needs_layout_passes=False,
                                             use_tc_tiling_on_sc=False),
        scratch_types=(
            pltpu.VMEM_SHARED((N_PAD, F), jnp.float32),
            pltpu.VMEM_SHARED((N_PAD, HP), jnp.float32),
            pltpu.VMEM((K,), jnp.int32),
            pltpu.VMEM((K,), jnp.int32),
            pltpu.VMEM((K, HP), jnp.float32),
            pltpu.VMEM((K, HP), jnp.float32),
            pltpu.VMEM((K, F), jnp.float32),
            pltpu.VMEM((K, HP), jnp.float32),
            pltpu.SemaphoreType.DMA,
            pltpu.SemaphoreType.DMA,
            pltpu.SemaphoreType.DMA,
        ),
    )


_sc_edge4 = _make_sc_edge(H, HC)
_sc_edge1 = _make_sc_edge(1, OUT)


def _tc_head(xp, W, Asd):
    Fin, Fout = W.shape

    def body(x_ref, w_ref, p_ref, h_ref, ab_ref):
        h = jnp.dot(x_ref[...], w_ref[...], preferred_element_type=jnp.float32)
        h_ref[...] = h
        ab_ref[...] = jnp.dot(h, p_ref[...], preferred_element_type=jnp.float32)

    return pl.pallas_call(
        body,
        grid=(GRID,),
        in_specs=[pl.BlockSpec((RB, Fin), lambda i: (i, 0)),
                  pl.BlockSpec((Fin, Fout), lambda i: (0, 0)),
                  pl.BlockSpec((Fout, HP), lambda i: (0, 0))],
        out_specs=[pl.BlockSpec((RB, Fout), lambda i: (i, 0)),
                   pl.BlockSpec((RB, HP), lambda i: (i, 0))],
        out_shape=[jax.ShapeDtypeStruct((N_PAD, Fout), jnp.float32),
                   jax.ShapeDtypeStruct((N_PAD, HP), jnp.float32)],
    )(xp, W, Asd)


def _tc_mid(oA, oB, dA, dB, expand, bias, W, Asd):
    Hh_in = expand.shape[0]
    Fin, Fout = W.shape

    def body(oa, ob, da, db, ex, bi, w, p_ref, h_ref, ab_ref):
        den = da[...] + db[...]
        dex = jnp.dot(den[:, :Hh_in], ex[...],
                      preferred_element_type=jnp.float32) + 1e-16
        g = _leaky((oa[...] + ob[...]) / dex + bi[...])
        h = jnp.dot(g, w[...], preferred_element_type=jnp.float32)
        h_ref[...] = h
        ab_ref[...] = jnp.dot(h, p_ref[...], preferred_element_type=jnp.float32)

    return pl.pallas_call(
        body,
        grid=(GRID,),
        in_specs=[pl.BlockSpec((RB, Fin), lambda i: (i, 0)),
                  pl.BlockSpec((RB, Fin), lambda i: (i, 0)),
                  pl.BlockSpec((RB, HP), lambda i: (i, 0)),
                  pl.BlockSpec((RB, HP), lambda i: (i, 0)),
                  pl.BlockSpec((Hh_in, Fin), lambda i: (0, 0)),
                  pl.BlockSpec((1, Fin), lambda i: (0, 0)),
                  pl.BlockSpec((Fin, Fout), lambda i: (0, 0)),
                  pl.BlockSpec((Fout, HP), lambda i: (0, 0))],
        out_specs=[pl.BlockSpec((RB, Fout), lambda i: (i, 0)),
                   pl.BlockSpec((RB, HP), lambda i: (i, 0))],
        out_shape=[jax.ShapeDtypeStruct((N_PAD, Fout), jnp.float32),
                   jax.ShapeDtypeStruct((N_PAD, HP), jnp.float32)],
    )(oA, oB, dA, dB, expand, bias, W, Asd)


def _tc_pool(oA, oB, dA, dB, bias, batch2d):
    def body(oa, ob, da, db, bi, bt, out_ref, acc, cnt):
        i = pl.program_id(0)
        den = (da[...] + db[...])[:, :1]
        dex = jnp.dot(den, jnp.ones((1, OUT), jnp.float32),
                      preferred_element_type=jnp.float32) + 1e-16
        h = _leaky((oa[...] + ob[...]) / dex + bi[...])
        onehot_t = (lax.broadcasted_iota(jnp.int32, (G, RB), 0)
                    == bt[...]).astype(jnp.float32)
        sums = jnp.dot(onehot_t, h, preferred_element_type=jnp.float32)
        counts = jnp.dot(onehot_t, jnp.ones((RB, 1), jnp.float32),
                         preferred_element_type=jnp.float32)

        @pl.when(i == 0)
        def _():
            acc[...] = jnp.zeros_like(acc)
            cnt[...] = jnp.zeros_like(cnt)

        acc[...] += sums
        cnt[...] += counts

        @pl.when(i == GRID - 1)
        def _():
            out_ref[...] = acc[...] / jnp.maximum(cnt[...], 1.0)

    return pl.pallas_call(
        body,
        grid=(GRID,),
        in_specs=[pl.BlockSpec((RB, OUT), lambda i: (i, 0)),
                  pl.BlockSpec((RB, OUT), lambda i: (i, 0)),
                  pl.BlockSpec((RB, HP), lambda i: (i, 0)),
                  pl.BlockSpec((RB, HP), lambda i: (i, 0)),
                  pl.BlockSpec((1, OUT), lambda i: (0, 0)),
                  pl.BlockSpec((1, RB), lambda i: (0, i))],
        out_specs=pl.BlockSpec((G, OUT), lambda i: (0, 0)),
        out_shape=jax.ShapeDtypeStruct((G, OUT), jnp.float32),
        scratch_shapes=[pltpu.VMEM((G, OUT), jnp.float32),
                        pltpu.VMEM((G, 1), jnp.float32)],
    )(oA, oB, dA, dB, bias, batch2d)


def _proj(a_s, a_d, Fout, Hh):
    if Hh > 1:
        mask = jnp.repeat(jnp.eye(Hh, dtype=jnp.float32), Fout // Hh, axis=0)
        As = a_s.reshape(Fout, 1) * mask
        Ad = a_d.reshape(Fout, 1) * mask
    else:
        As = a_s.reshape(Fout, 1)
        Ad = a_d.reshape(Fout, 1)
    z = jnp.zeros((Fout, 8 - Hh), jnp.float32)
    return jnp.concatenate([As, z, Ad, z], axis=1)


def kernel(x, edge_index, batch, W0, a_s0, a_d0, b0,
           W1, a_s1, a_d1, b1, W2, a_s2, a_d2, b2):
    f32 = jnp.float32
    ei = edge_index.astype(jnp.int32)
    loops = jnp.arange(N, dtype=jnp.int32)
    pad_e = jnp.full((E_PAD - E1,), N, jnp.int32)
    src = jnp.concatenate([ei[0], loops, pad_e])
    dst = jnp.concatenate([ei[1], loops, pad_e])
    xp = jnp.pad(x.astype(f32), ((0, N_PAD - N), (0, 0)))
    batch2d = jnp.pad(batch.astype(jnp.int32), (0, N_PAD - N),
                      constant_values=G).reshape(1, N_PAD)

    mask4 = jnp.repeat(jnp.eye(H, dtype=f32), C, axis=0)
    expand4 = mask4.T
    P0 = _proj(a_s0, a_d0, HC, H)
    P1 = _proj(a_s1, a_d1, HC, H)
    P2 = _proj(a_s2, a_d2, OUT, 1)

    h0, ab0 = _tc_head(xp, W0, P0)
    o0, den0 = _sc_edge4(src, dst, ab0, h0)
    h1, ab1 = _tc_mid(o0[0], o0[1], den0[0], den0[1], expand4,
                      b0.reshape(1, HC), W1, P1)
    o1, den1 = _sc_edge4(src, dst, ab1, h1)
    h2, ab2 = _tc_mid(o1[0], o1[1], den1[0], den1[1], expand4,
                      b1.reshape(1, HC), W2, P2)
    o2, den2 = _sc_edge1(src, dst, ab2, h2)
    return _tc_pool(o2[0], o2[1], den2[0], den2[1],
                    b2.reshape(1, OUT), batch2d)


# speedup vs baseline: 42.9216x; 1.0426x over previous
import jax
import jax.numpy as jnp
from jax import lax
from jax.experimental import pallas as pl
from jax.experimental.pallas import tpu as pltpu
from jax.experimental.pallas import tpu_sc as plsc

N = 10000
D = 128
H = 4
C = 32
HC = H * C
OUT = 64
G = 64
E = 320000
E1 = E + N
NEG = 0.2

NC = 2
NS = 16
NW = NC * NS
K = 96
EW = 10368
E_PAD = NW * EW
N_PAD = 10240
SLICE = N_PAD // NS
HP = 16
RB = 1280
GRID = N_PAD // RB


def _leaky(v):
    return jnp.where(v > 0, v, NEG * v)


def _make_sc_edge(Hh, F):
    CW = F // Hh
    CH = EW // K
    EPI = 16 // Hh

    def body(src_hbm, dst_hbm, ab_hbm, h_hbm, out_hbm, den_hbm,
             acc_out, acc_den, src_v, dst_v0, dst_v1, sa, sb,
             hrows0, hrows1, s_rows0, s_rows1,
             sem_h, sem_a, sem_b, sem_s0, sem_s1):
        cid = lax.axis_index("c")
        sid = lax.axis_index("s")
        wid = sid * NC + cid
        lanes = lax.iota(jnp.int32, 16)
        dst_v = (dst_v0, dst_v1)
        hrows = (hrows0, hrows1)
        s_rows = (s_rows0, s_rows1)
        sem_s = (sem_s0, sem_s1)

        def zr(k_, _):
            for c0 in range(0, F, 16):
                hrows0[k_, pl.ds(c0, 16)] = jnp.zeros((16,), jnp.float32)
            return 0
        lax.fori_loop(0, K, zr, 0)

        def zs(i, _):
            plsc.store_scatter(s_rows0, [jnp.full((16,), i, jnp.int32), lanes],
                               jnp.zeros((16,), jnp.float32))
            return 0
        lax.fori_loop(0, K, zs, 0)

        base = sid * SLICE
        zoff = 0
        while zoff < SLICE:
            zn = min(K, SLICE - zoff)
            pltpu.sync_copy(hrows0.at[pl.ds(0, zn)],
                            acc_out.at[pl.ds(base + zoff, zn)])
            pltpu.sync_copy(s_rows0.at[pl.ds(0, zn)],
                            acc_den.at[pl.ds(base + zoff, zn)])
            zoff += zn
        plsc.subcore_barrier()

        ebase = wid * EW

        def pair(i, _):
            for b in (0, 1):
                c = 2 * i + b

                @pl.when(i > 0)
                def _():
                    pltpu.make_async_copy(
                        hrows[b], acc_out.at[dst_v[b]], sem_s[b]).wait()
                    pltpu.make_async_copy(
                        s_rows[b], acc_den.at[dst_v[b]], sem_s[b]).wait()

                off = ebase + c * K
                pltpu.sync_copy(src_hbm.at[pl.ds(off, K)], src_v)
                pltpu.sync_copy(dst_hbm.at[pl.ds(off, K)], dst_v[b])
                gh = pltpu.async_copy(h_hbm.at[src_v], hrows[b], sem_h)
                ga = pltpu.async_copy(ab_hbm.at[src_v], sa, sem_a)
                gb = pltpu.async_copy(ab_hbm.at[dst_v[b]], sb, sem_b)
                ga.wait()
                gb.wait()

                def sgrp(j_, _):
                    rows = j_ * EPI + lanes // Hh
                    cols = lanes % Hh
                    a = plsc.load_gather(sa, [rows, cols])
                    bb = plsc.load_gather(sb, [rows, cols + 8])
                    sv = jnp.exp(_leaky(a + bb))
                    plsc.store_scatter(s_rows[b], [rows, cols], sv)
                    return 0
                lax.fori_loop(0, K // EPI, sgrp, 0, unroll=4)
                gh.wait()

                def erow(k_, _):
                    row_idx = jnp.full((16,), k_, jnp.int32)
                    for hh in range(Hh):
                        m = plsc.load_gather(
                            s_rows[b], [row_idx, jnp.full((16,), hh, jnp.int32)])
                        for q in range(CW // 16):
                            c0 = hh * CW + q * 16
                            hrows[b][k_, pl.ds(c0, 16)] = (
                                hrows[b][k_, pl.ds(c0, 16)] * m)
                    return 0
                lax.fori_loop(0, K, erow, 0, unroll=4)

                pltpu.async_copy(hrows[b], acc_out.at[dst_v[b]], sem_s[b],
                                 add=True)
                pltpu.async_copy(s_rows[b], acc_den.at[dst_v[b]], sem_s[b],
                                 add=True)
            return 0
        lax.fori_loop(0, CH // 2, pair, 0)
        for b in (0, 1):
            pltpu.make_async_copy(hrows[b], acc_out.at[dst_v[b]],
                                  sem_s[b]).wait()
            pltpu.make_async_copy(s_rows[b], acc_den.at[dst_v[b]],
                                  sem_s[b]).wait()
        plsc.subcore_barrier()

        pltpu.sync_copy(acc_out.at[pl.ds(base, SLICE)],
                        out_hbm.at[cid, pl.ds(base, SLICE)])
        pltpu.sync_copy(acc_den.at[pl.ds(base, SLICE)],
                        den_hbm.at[cid, pl.ds(base, SLICE)])

    return pl.kernel(
        body,
        out_type=(jax.ShapeDtypeStruct((NC, N_PAD, F), jnp.float32),
                  jax.ShapeDtypeStruct((NC, N_PAD, HP), jnp.float32)),
        mesh=plsc.VectorSubcoreMesh(core_axis_name="c", subcore_axis_name="s"),
        compiler_params=pltpu.CompilerParams(needs_layout_passes=False,
                                             use_tc_tiling_on_sc=False),
        scratch_types=(
            pltpu.VMEM_SHARED((N_PAD, F), jnp.float32),
            pltpu.VMEM_SHARED((N_PAD, HP), jnp.float32),
            pltpu.VMEM((K,), jnp.int32),
            pltpu.VMEM((K,), jnp.int32),
            pltpu.VMEM((K,), jnp.int32),
            pltpu.VMEM((K, HP), jnp.float32),
            pltpu.VMEM((K, HP), jnp.float32),
            pltpu.VMEM((K, F), jnp.float32),
            pltpu.VMEM((K, F), jnp.float32),
            pltpu.VMEM((K, HP), jnp.float32),
            pltpu.VMEM((K, HP), jnp.float32),
            pltpu.SemaphoreType.DMA,
            pltpu.SemaphoreType.DMA,
            pltpu.SemaphoreType.DMA,
            pltpu.SemaphoreType.DMA,
            pltpu.SemaphoreType.DMA,
        ),
    )


_sc_edge4 = _make_sc_edge(H, HC)
_sc_edge1 = _make_sc_edge(1, OUT)


def _tc_head(xp, W, Asd):
    Fin, Fout = W.shape

    def body(x_ref, w_ref, p_ref, h_ref, ab_ref):
        h = jnp.dot(x_ref[...], w_ref[...], preferred_element_type=jnp.float32)
        h_ref[...] = h
        ab_ref[...] = jnp.dot(h, p_ref[...], preferred_element_type=jnp.float32)

    return pl.pallas_call(
        body,
        grid=(GRID,),
        in_specs=[pl.BlockSpec((RB, Fin), lambda i: (i, 0)),
                  pl.BlockSpec((Fin, Fout), lambda i: (0, 0)),
                  pl.BlockSpec((Fout, HP), lambda i: (0, 0))],
        out_specs=[pl.BlockSpec((RB, Fout), lambda i: (i, 0)),
                   pl.BlockSpec((RB, HP), lambda i: (i, 0))],
        out_shape=[jax.ShapeDtypeStruct((N_PAD, Fout), jnp.float32),
                   jax.ShapeDtypeStruct((N_PAD, HP), jnp.float32)],
    )(xp, W, Asd)


def _tc_mid(oA, oB, dA, dB, expand, bias, W, Asd):
    Hh_in = expand.shape[0]
    Fin, Fout = W.shape

    def body(oa, ob, da, db, ex, bi, w, p_ref, h_ref, ab_ref):
        den = da[...] + db[...]
        dex = jnp.dot(den[:, :Hh_in], ex[...],
                      preferred_element_type=jnp.float32) + 1e-16
        g = _leaky((oa[...] + ob[...]) / dex + bi[...])
        h = jnp.dot(g, w[...], preferred_element_type=jnp.float32)
        h_ref[...] = h
        ab_ref[...] = jnp.dot(h, p_ref[...], preferred_element_type=jnp.float32)

    return pl.pallas_call(
        body,
        grid=(GRID,),
        in_specs=[pl.BlockSpec((RB, Fin), lambda i: (i, 0)),
                  pl.BlockSpec((RB, Fin), lambda i: (i, 0)),
                  pl.BlockSpec((RB, HP), lambda i: (i, 0)),
                  pl.BlockSpec((RB, HP), lambda i: (i, 0)),
                  pl.BlockSpec((Hh_in, Fin), lambda i: (0, 0)),
                  pl.BlockSpec((1, Fin), lambda i: (0, 0)),
                  pl.BlockSpec((Fin, Fout), lambda i: (0, 0)),
                  pl.BlockSpec((Fout, HP), lambda i: (0, 0))],
        out_specs=[pl.BlockSpec((RB, Fout), lambda i: (i, 0)),
                   pl.BlockSpec((RB, HP), lambda i: (i, 0))],
        out_shape=[jax.ShapeDtypeStruct((N_PAD, Fout), jnp.float32),
                   jax.ShapeDtypeStruct((N_PAD, HP), jnp.float32)],
    )(oA, oB, dA, dB, expand, bias, W, Asd)


def _tc_pool(oA, oB, dA, dB, bias, batch2d):
    def body(oa, ob, da, db, bi, bt, out_ref, acc, cnt):
        i = pl.program_id(0)
        den = (da[...] + db[...])[:, :1]
        dex = jnp.dot(den, jnp.ones((1, OUT), jnp.float32),
                      preferred_element_type=jnp.float32) + 1e-16
        h = _leaky((oa[...] + ob[...]) / dex + bi[...])
        onehot_t = (lax.broadcasted_iota(jnp.int32, (G, RB), 0)
                    == bt[...]).astype(jnp.float32)
        sums = jnp.dot(onehot_t, h, preferred_element_type=jnp.float32)
        counts = jnp.dot(onehot_t, jnp.ones((RB, 1), jnp.float32),
                         preferred_element_type=jnp.float32)

        @pl.when(i == 0)
        def _():
            acc[...] = jnp.zeros_like(acc)
            cnt[...] = jnp.zeros_like(cnt)

        acc[...] += sums
        cnt[...] += counts

        @pl.when(i == GRID - 1)
        def _():
            out_ref[...] = acc[...] / jnp.maximum(cnt[...], 1.0)

    return pl.pallas_call(
        body,
        grid=(GRID,),
        in_specs=[pl.BlockSpec((RB, OUT), lambda i: (i, 0)),
                  pl.BlockSpec((RB, OUT), lambda i: (i, 0)),
                  pl.BlockSpec((RB, HP), lambda i: (i, 0)),
                  pl.BlockSpec((RB, HP), lambda i: (i, 0)),
                  pl.BlockSpec((1, OUT), lambda i: (0, 0)),
                  pl.BlockSpec((1, RB), lambda i: (0, i))],
        out_specs=pl.BlockSpec((G, OUT), lambda i: (0, 0)),
        out_shape=jax.ShapeDtypeStruct((G, OUT), jnp.float32),
        scratch_shapes=[pltpu.VMEM((G, OUT), jnp.float32),
                        pltpu.VMEM((G, 1), jnp.float32)],
    )(oA, oB, dA, dB, bias, batch2d)


def _proj(a_s, a_d, Fout, Hh):
    if Hh > 1:
        mask = jnp.repeat(jnp.eye(Hh, dtype=jnp.float32), Fout // Hh, axis=0)
        As = a_s.reshape(Fout, 1) * mask
        Ad = a_d.reshape(Fout, 1) * mask
    else:
        As = a_s.reshape(Fout, 1)
        Ad = a_d.reshape(Fout, 1)
    z = jnp.zeros((Fout, 8 - Hh), jnp.float32)
    return jnp.concatenate([As, z, Ad, z], axis=1)


def kernel(x, edge_index, batch, W0, a_s0, a_d0, b0,
           W1, a_s1, a_d1, b1, W2, a_s2, a_d2, b2):
    f32 = jnp.float32
    ei = edge_index.astype(jnp.int32)
    loops = jnp.arange(N, dtype=jnp.int32)
    pad_e = jnp.full((E_PAD - E1,), N, jnp.int32)
    src = jnp.concatenate([ei[0], loops, pad_e])
    dst = jnp.concatenate([ei[1], loops, pad_e])
    xp = jnp.pad(x.astype(f32), ((0, N_PAD - N), (0, 0)))
    batch2d = jnp.pad(batch.astype(jnp.int32), (0, N_PAD - N),
                      constant_values=G).reshape(1, N_PAD)

    mask4 = jnp.repeat(jnp.eye(H, dtype=f32), C, axis=0)
    expand4 = mask4.T
    P0 = _proj(a_s0, a_d0, HC, H)
    P1 = _proj(a_s1, a_d1, HC, H)
    P2 = _proj(a_s2, a_d2, OUT, 1)

    h0, ab0 = _tc_head(xp, W0, P0)
    o0, den0 = _sc_edge4(src, dst, ab0, h0)
    h1, ab1 = _tc_mid(o0[0], o0[1], den0[0], den0[1], expand4,
                      b0.reshape(1, HC), W1, P1)
    o1, den1 = _sc_edge4(src, dst, ab1, h1)
    h2, ab2 = _tc_mid(o1[0], o1[1], den1[0], den1[1], expand4,
                      b1.reshape(1, HC), W2, P2)
    o2, den2 = _sc_edge1(src, dst, ab2, h2)
    return _tc_pool(o2[0], o2[1], den2[0], den2[1],
                    b2.reshape(1, OUT), batch2d)


# speedup vs baseline: 63.2479x; 1.4736x over previous
import jax
import jax.numpy as jnp
from jax import lax
from jax.experimental import pallas as pl
from jax.experimental.pallas import tpu as pltpu
from jax.experimental.pallas import tpu_sc as plsc

N = 10000
D = 128
H = 4
C = 32
HC = H * C
OUT = 64
G = 64
E = 320000
E1 = E + N
NEG = 0.2

NC = 2
NS = 16
NW = NC * NS
K = 96
EW = 10368
E_PAD = NW * EW
N_PAD = 10240
SLICE = N_PAD // NS
HP = 16
RB = 1280
GRID = N_PAD // RB


def _leaky(v):
    return jnp.where(v > 0, v, NEG * v)


def _make_sc_edge(Hh, F):
    CW = F // Hh
    CH = EW // K
    EPI = 16 // Hh

    SB = 18

    def body(src_hbm, dst_hbm, ab_hbm, h_hbm, out_hbm, den_hbm,
             acc_out, acc_den, src_all, dst_all, sa0, sa1, sb0, sb1,
             hrows0, hrows1, s_rows0, s_rows1,
             sem_h0, sem_h1, sem_a0, sem_a1, sem_b0, sem_b1,
             sem_s0, sem_s1):
        cid = lax.axis_index("c")
        sid = lax.axis_index("s")
        wid = sid * NC + cid
        lanes = lax.iota(jnp.int32, 16)
        sa = (sa0, sa1)
        sbuf = (sb0, sb1)
        hrows = (hrows0, hrows1)
        s_rows = (s_rows0, s_rows1)
        sem_h = (sem_h0, sem_h1)
        sem_a = (sem_a0, sem_a1)
        sem_b = (sem_b0, sem_b1)
        sem_s = (sem_s0, sem_s1)

        def zr(k_, _):
            for c0 in range(0, F, 16):
                hrows0[k_, pl.ds(c0, 16)] = jnp.zeros((16,), jnp.float32)
            return 0
        lax.fori_loop(0, K, zr, 0)

        def zs(i, _):
            plsc.store_scatter(s_rows0, [jnp.full((16,), i, jnp.int32), lanes],
                               jnp.zeros((16,), jnp.float32))
            return 0
        lax.fori_loop(0, K, zs, 0)

        base = sid * SLICE
        zoff = 0
        while zoff < SLICE:
            zn = min(K, SLICE - zoff)
            pltpu.sync_copy(hrows0.at[pl.ds(0, zn)],
                            acc_out.at[pl.ds(base + zoff, zn)])
            pltpu.sync_copy(s_rows0.at[pl.ds(0, zn)],
                            acc_den.at[pl.ds(base + zoff, zn)])
            zoff += zn
        plsc.subcore_barrier()

        cbase = wid * CH

        def issue_gathers(j, b):
            pltpu.async_copy(h_hbm.at[src_all.at[j]], hrows[b], sem_h[b])
            pltpu.async_copy(ab_hbm.at[src_all.at[j]], sa[b], sem_a[b])
            pltpu.async_copy(ab_hbm.at[dst_all.at[j]], sbuf[b], sem_b[b])

        def drain_scatters(b):
            pltpu.make_async_copy(hrows[b], acc_out.at[dst_all.at[0]],
                                  sem_s[b]).wait()
            pltpu.make_async_copy(s_rows[b], acc_den.at[dst_all.at[0]],
                                  sem_s[b]).wait()

        def superblock(s_i, _):
            srow = cbase + s_i * SB
            pltpu.sync_copy(src_hbm.at[pl.ds(srow, SB)], src_all)
            pltpu.sync_copy(dst_hbm.at[pl.ds(srow, SB)], dst_all)
            issue_gathers(0, 0)

            def pair(p, _):
                for b in (0, 1):
                    j = 2 * p + b
                    nb = 1 - b
                    pltpu.make_async_copy(ab_hbm.at[src_all.at[j]], sa[b],
                                          sem_a[b]).wait()
                    pltpu.make_async_copy(ab_hbm.at[dst_all.at[j]], sbuf[b],
                                          sem_b[b]).wait()

                    def sgrp(j_, _):
                        rows = j_ * EPI + lanes // Hh
                        cols = lanes % Hh
                        a = plsc.load_gather(sa[b], [rows, cols])
                        bb = plsc.load_gather(sbuf[b], [rows, cols + 8])
                        sv = jnp.exp(_leaky(a + bb))
                        plsc.store_scatter(s_rows[b], [rows, cols], sv)
                        return 0
                    lax.fori_loop(0, K // EPI, sgrp, 0, unroll=4)

                    if b == 0:
                        @pl.when(p > 0)
                        def _():
                            drain_scatters(nb)
                        issue_gathers(j + 1, nb)
                    else:
                        drain_scatters(nb)

                        @pl.when(p < SB // 2 - 1)
                        def _():
                            issue_gathers(j + 1, nb)

                    pltpu.make_async_copy(h_hbm.at[src_all.at[j]], hrows[b],
                                          sem_h[b]).wait()

                    def erow(k_, _):
                        row_idx = jnp.full((16,), k_, jnp.int32)
                        for hh in range(Hh):
                            m = plsc.load_gather(
                                s_rows[b],
                                [row_idx, jnp.full((16,), hh, jnp.int32)])
                            for q in range(CW // 16):
                                c0 = hh * CW + q * 16
                                hrows[b][k_, pl.ds(c0, 16)] = (
                                    hrows[b][k_, pl.ds(c0, 16)] * m)
                        return 0
                    lax.fori_loop(0, K, erow, 0, unroll=4)

                    pltpu.async_copy(hrows[b], acc_out.at[dst_all.at[j]],
                                     sem_s[b], add=True)
                    pltpu.async_copy(s_rows[b], acc_den.at[dst_all.at[j]],
                                     sem_s[b], add=True)
                return 0
            lax.fori_loop(0, SB // 2, pair, 0)
            drain_scatters(1)
            return 0
        lax.fori_loop(0, CH // SB, superblock, 0)
        plsc.subcore_barrier()

        pltpu.sync_copy(acc_out.at[pl.ds(base, SLICE)],
                        out_hbm.at[cid, pl.ds(base, SLICE)])
        pltpu.sync_copy(acc_den.at[pl.ds(base, SLICE)],
                        den_hbm.at[cid, pl.ds(base, SLICE)])

    return pl.kernel(
        body,
        out_type=(jax.ShapeDtypeStruct((NC, N_PAD, F), jnp.float32),
                  jax.ShapeDtypeStruct((NC, N_PAD, HP), jnp.float32)),
        mesh=plsc.VectorSubcoreMesh(core_axis_name="c", subcore_axis_name="s"),
        compiler_params=pltpu.CompilerParams(needs_layout_passes=False,
                                             use_tc_tiling_on_sc=False),
        scratch_types=(
            pltpu.VMEM_SHARED((N_PAD, F), jnp.float32),
            pltpu.VMEM_SHARED((N_PAD, HP), jnp.float32),
            pltpu.VMEM((SB, K), jnp.int32),
            pltpu.VMEM((SB, K), jnp.int32),
            pltpu.VMEM((K, HP), jnp.float32),
            pltpu.VMEM((K, HP), jnp.float32),
            pltpu.VMEM((K, HP), jnp.float32),
            pltpu.VMEM((K, HP), jnp.float32),
            pltpu.VMEM((K, F), jnp.float32),
            pltpu.VMEM((K, F), jnp.float32),
            pltpu.VMEM((K, HP), jnp.float32),
            pltpu.VMEM((K, HP), jnp.float32),
            pltpu.SemaphoreType.DMA,
            pltpu.SemaphoreType.DMA,
            pltpu.SemaphoreType.DMA,
            pltpu.SemaphoreType.DMA,
            pltpu.SemaphoreType.DMA,
            pltpu.SemaphoreType.DMA,
            pltpu.SemaphoreType.DMA,
            pltpu.SemaphoreType.DMA,
        ),
    )


_sc_edge4 = _make_sc_edge(H, HC)
_sc_edge1 = _make_sc_edge(1, OUT)


def _tc_head(xp, W, Asd):
    Fin, Fout = W.shape

    def body(x_ref, w_ref, p_ref, h_ref, ab_ref):
        h = jnp.dot(x_ref[...], w_ref[...], preferred_element_type=jnp.float32)
        h_ref[...] = h
        ab_ref[...] = jnp.dot(h, p_ref[...], preferred_element_type=jnp.float32)

    return pl.pallas_call(
        body,
        grid=(GRID,),
        in_specs=[pl.BlockSpec((RB, Fin), lambda i: (i, 0)),
                  pl.BlockSpec((Fin, Fout), lambda i: (0, 0)),
                  pl.BlockSpec((Fout, HP), lambda i: (0, 0))],
        out_specs=[pl.BlockSpec((RB, Fout), lambda i: (i, 0)),
                   pl.BlockSpec((RB, HP), lambda i: (i, 0))],
        out_shape=[jax.ShapeDtypeStruct((N_PAD, Fout), jnp.float32),
                   jax.ShapeDtypeStruct((N_PAD, HP), jnp.float32)],
    )(xp, W, Asd)


def _tc_mid(oA, oB, dA, dB, expand, bias, W, Asd):
    Hh_in = expand.shape[0]
    Fin, Fout = W.shape

    def body(oa, ob, da, db, ex, bi, w, p_ref, h_ref, ab_ref):
        den = da[...] + db[...]
        dex = jnp.dot(den[:, :Hh_in], ex[...],
                      preferred_element_type=jnp.float32) + 1e-16
        g = _leaky((oa[...] + ob[...]) / dex + bi[...])
        h = jnp.dot(g, w[...], preferred_element_type=jnp.float32)
        h_ref[...] = h
        ab_ref[...] = jnp.dot(h, p_ref[...], preferred_element_type=jnp.float32)

    return pl.pallas_call(
        body,
        grid=(GRID,),
        in_specs=[pl.BlockSpec((RB, Fin), lambda i: (i, 0)),
                  pl.BlockSpec((RB, Fin), lambda i: (i, 0)),
                  pl.BlockSpec((RB, HP), lambda i: (i, 0)),
                  pl.BlockSpec((RB, HP), lambda i: (i, 0)),
                  pl.BlockSpec((Hh_in, Fin), lambda i: (0, 0)),
                  pl.BlockSpec((1, Fin), lambda i: (0, 0)),
                  pl.BlockSpec((Fin, Fout), lambda i: (0, 0)),
                  pl.BlockSpec((Fout, HP), lambda i: (0, 0))],
        out_specs=[pl.BlockSpec((RB, Fout), lambda i: (i, 0)),
                   pl.BlockSpec((RB, HP), lambda i: (i, 0))],
        out_shape=[jax.ShapeDtypeStruct((N_PAD, Fout), jnp.float32),
                   jax.ShapeDtypeStruct((N_PAD, HP), jnp.float32)],
    )(oA, oB, dA, dB, expand, bias, W, Asd)


def _tc_pool(oA, oB, dA, dB, bias, batch2d):
    def body(oa, ob, da, db, bi, bt, out_ref, acc, cnt):
        i = pl.program_id(0)
        den = (da[...] + db[...])[:, :1]
        dex = jnp.dot(den, jnp.ones((1, OUT), jnp.float32),
                      preferred_element_type=jnp.float32) + 1e-16
        h = _leaky((oa[...] + ob[...]) / dex + bi[...])
        onehot_t = (lax.broadcasted_iota(jnp.int32, (G, RB), 0)
                    == bt[...]).astype(jnp.float32)
        sums = jnp.dot(onehot_t, h, preferred_element_type=jnp.float32)
        counts = jnp.dot(onehot_t, jnp.ones((RB, 1), jnp.float32),
                         preferred_element_type=jnp.float32)

        @pl.when(i == 0)
        def _():
            acc[...] = jnp.zeros_like(acc)
            cnt[...] = jnp.zeros_like(cnt)

        acc[...] += sums
        cnt[...] += counts

        @pl.when(i == GRID - 1)
        def _():
            out_ref[...] = acc[...] / jnp.maximum(cnt[...], 1.0)

    return pl.pallas_call(
        body,
        grid=(GRID,),
        in_specs=[pl.BlockSpec((RB, OUT), lambda i: (i, 0)),
                  pl.BlockSpec((RB, OUT), lambda i: (i, 0)),
                  pl.BlockSpec((RB, HP), lambda i: (i, 0)),
                  pl.BlockSpec((RB, HP), lambda i: (i, 0)),
                  pl.BlockSpec((1, OUT), lambda i: (0, 0)),
                  pl.BlockSpec((1, RB), lambda i: (0, i))],
        out_specs=pl.BlockSpec((G, OUT), lambda i: (0, 0)),
        out_shape=jax.ShapeDtypeStruct((G, OUT), jnp.float32),
        scratch_shapes=[pltpu.VMEM((G, OUT), jnp.float32),
                        pltpu.VMEM((G, 1), jnp.float32)],
    )(oA, oB, dA, dB, bias, batch2d)


def _proj(a_s, a_d, Fout, Hh):
    if Hh > 1:
        mask = jnp.repeat(jnp.eye(Hh, dtype=jnp.float32), Fout // Hh, axis=0)
        As = a_s.reshape(Fout, 1) * mask
        Ad = a_d.reshape(Fout, 1) * mask
    else:
        As = a_s.reshape(Fout, 1)
        Ad = a_d.reshape(Fout, 1)
    z = jnp.zeros((Fout, 8 - Hh), jnp.float32)
    return jnp.concatenate([As, z, Ad, z], axis=1)


def kernel(x, edge_index, batch, W0, a_s0, a_d0, b0,
           W1, a_s1, a_d1, b1, W2, a_s2, a_d2, b2):
    f32 = jnp.float32
    ei = edge_index.astype(jnp.int32)
    loops = jnp.arange(N, dtype=jnp.int32)
    pad_e = jnp.full((E_PAD - E1,), N, jnp.int32)
    src = jnp.concatenate([ei[0], loops, pad_e]).reshape(E_PAD // K, K)
    dst = jnp.concatenate([ei[1], loops, pad_e]).reshape(E_PAD // K, K)
    xp = jnp.pad(x.astype(f32), ((0, N_PAD - N), (0, 0)))
    batch2d = jnp.pad(batch.astype(jnp.int32), (0, N_PAD - N),
                      constant_values=G).reshape(1, N_PAD)

    mask4 = jnp.repeat(jnp.eye(H, dtype=f32), C, axis=0)
    expand4 = mask4.T
    P0 = _proj(a_s0, a_d0, HC, H)
    P1 = _proj(a_s1, a_d1, HC, H)
    P2 = _proj(a_s2, a_d2, OUT, 1)

    h0, ab0 = _tc_head(xp, W0, P0)
    o0, den0 = _sc_edge4(src, dst, ab0, h0)
    h1, ab1 = _tc_mid(o0[0], o0[1], den0[0], den0[1], expand4,
                      b0.reshape(1, HC), W1, P1)
    o1, den1 = _sc_edge4(src, dst, ab1, h1)
    h2, ab2 = _tc_mid(o1[0], o1[1], den1[0], den1[1], expand4,
                      b1.reshape(1, HC), W2, P2)
    o2, den2 = _sc_edge1(src, dst, ab2, h2)
    return _tc_pool(o2[0], o2[1], den2[0], den2[1],
                    b2.reshape(1, OUT), batch2d)
